# baseline (speedup 1.0000x reference)
import jax
import jax.numpy as jnp
from jax import lax
from jax.experimental import pallas as pl
from jax.experimental.pallas import tpu as pltpu

_N = 64
_D = 128
_NEG = -1e30

_DN_STD = (((1,), (0,)), ((), ()))
_DN_LAST = (((1,), (1,)), ((), ()))
_DN_S0 = (((0,), (0,)), ((), ()))


def _gen_body(z_ref, w1_ref, wes_ref, gwcat_ref, garm_ref, bias_ref,
              dbig_ref, der_ref, consts_ref, out_ref, feat_s):
    z = z_ref[...]
    w1z = w1_ref[:, :_D]
    w1s = w1_ref[:, _D:]
    wez = wes_ref[0:1, :]
    wes = wes_ref[1:2, :]
    garm = garm_ref[...]
    bias_mean = bias_ref[...]
    w2lg = dbig_ref[0:1, 3 * _D:3 * _D + 4]
    b2lg = dbig_ref[1:2, 3 * _D:3 * _D + 4]
    bias_lg = dbig_ref[2:3, 3 * _D:3 * _D + 4]
    w2ft = dbig_ref[0:1, :3 * _D]
    b2ft = dbig_ref[1:2, :3 * _D]
    w2er = der_ref[:, 0:1]
    b2er = der_ref[:, 1:2]
    bias_er = der_ref[:, 2:3]
    b1s = consts_ref[0, 0]
    bes = consts_ref[0, 1]
    w2we = consts_ref[0, 2]
    b2we = consts_ref[0, 3]

    row_i = lax.broadcasted_iota(jnp.int32, (_N, 1), 0)
    row_f = row_i.astype(jnp.float32)
    lane_i = lax.broadcasted_iota(jnp.int32, (1, _N), 1)
    d_row_f = lane_i.astype(jnp.float32)

    zw1 = jnp.sum(z * w1z, axis=1, keepdims=True) + b1s
    zwe = jnp.sum(z * wez, axis=1, keepdims=True) + bes

    def tail(bg_lg, erg, s3, n2, kcol):
        tpre = zw1 + jnp.sum(s3 * w1s, axis=1, keepdims=True)
        stop = jnp.logical_or(tpre[0, 0] <= 0.0, n2 >= _N)
        tok = jnp.maximum(tpre, 0.0)
        lg2 = bg_lg + jnp.where(row_i == n2, tok * w2lg + b2lg, 0.0)
        er2 = erg + jnp.where(lane_i == n2, tok * w2er + b2er, 0.0)
        feat_s[pl.ds(jnp.minimum(n2, _N - 1), 1), :] = tok * w2ft + b2ft
        cbase = zwe + jnp.sum(s3 * wes, axis=1, keepdims=True) \
            + tok * w2we + b2we
        te = lg2[:, 3:4] + cbase
        brk = te < 1e-4
        cand = jnp.where(brk, row_f, jnp.float32(_N))
        knew = jnp.minimum(jnp.min(cand, axis=0, keepdims=True),
                           (n2 + 1).astype(jnp.float32))
        kcol2 = jnp.where(row_i == n2, knew, kcol)
        return kcol2, lg2, er2, stop

    out_ref[...] = jnp.where(row_i == 0, bias_mean, 0.0)
    feat_s[...] = jnp.where(row_i == 0, dbig_ref[2:3, :3 * _D], 0.0)
    lg0 = jnp.where(row_i == 0, bias_lg, 0.0)
    er0 = jnp.where(lane_i == 0, bias_er, 0.0)
    k0 = jnp.zeros((_N, 1), jnp.float32)
    k1, lg1, er1, stop1 = tail(lg0, er0, bias_mean, jnp.int32(1), k0)

    def cond(c):
        return jnp.logical_not(c[4])

    def body(c):
        kcol, lg, er3, n, _ = c
        mask = d_row_f < kcol
        n2f = (n + 1).astype(jnp.float32)
        acc = jnp.zeros((_N, _D), jnp.float32)
        for head in range(3):
            epre = lg[:, head:head + 1] + er3[head:head + 1, :]
            e = jnp.where(epre >= 0, epre, 0.2 * epre)
            em = jnp.where(mask, e, _NEG)
            m = jnp.max(em, axis=0, keepdims=True)
            m = jnp.where(m > 0.1 * _NEG, m, 0.0)
            ex = jnp.exp(em - m)
            denom = jnp.sum(ex, axis=0, keepdims=True)
            dsafe = jnp.where(denom > 0, denom, 1.0)
            alpha = ex / dsafe
            fh = feat_s[:, head * _D:(head + 1) * _D]
            acc = acc + lax.dot_general(alpha, fh, _DN_S0,
                                        preferred_element_type=jnp.float32)
        hnew = acc * (1.0 / 3.0) + bias_mean
        hg = jnp.where(row_f < n2f, hnew, 0.0)
        s3 = jnp.sum(hg, axis=0, keepdims=True) / n2f
        out_ref[...] = hg
        bg = lax.dot_general(hg, gwcat_ref[...], _DN_STD,
                             preferred_element_type=jnp.float32)
        erg = lax.dot_general(garm, hg, _DN_LAST,
                              preferred_element_type=jnp.float32)
        feat_s[...] = bg[:, :3 * _D]
        kcol2, lg2, er2, stop = tail(bg[:, 3 * _D:3 * _D + 4], erg, s3,
                                     n + 1, kcol)
        return (kcol2, lg2, er2, n + 1, stop)

    lax.while_loop(cond, body, (k1, lg1, er1, jnp.int32(1), stop1))


def kernel(z, W1, b1, W2, b2, We, be, gat_W, gat_b, attn_l, attn_r):
    f32 = jnp.float32
    al3 = attn_l.reshape(3, _D).astype(f32)
    ar3 = attn_r.reshape(3, _D).astype(f32)
    gw3 = gat_W.astype(f32).reshape(3, _D, _D)
    galmT = jnp.einsum('hc,hck->hk', al3, gw3)
    garm = jnp.einsum('hc,hck->hk', ar3, gw3)
    we4_ = We.reshape(4, _D).astype(f32)
    gwcat = jnp.concatenate([
        gat_W.astype(f32),
        galmT,
        we4_[3:4, :],
        jnp.zeros((512 - 384 - 4, _D), f32),
    ], axis=0).T
    gb3 = gat_b.reshape(3, _D).astype(f32)
    bias_mean = jnp.mean(gb3, axis=0, keepdims=True)
    w2row = W2.reshape(1, _D).astype(f32)
    b2r = b2.reshape(1, _D).astype(f32)
    gens = jnp.concatenate([w2row, b2r, bias_mean], axis=0)
    dbig = gens @ gwcat
    der = lax.dot_general(garm, gens, _DN_LAST)
    consts = jnp.stack([
        b1.reshape(()).astype(f32),
        be.reshape(()).astype(f32),
        jnp.sum(w2row[0] * we4_[2]),
        jnp.sum(b2r[0] * we4_[2]),
    ]).reshape(1, 4)
    vmem = pl.BlockSpec(memory_space=pltpu.VMEM)
    smem = pl.BlockSpec(memory_space=pltpu.SMEM)
    return pl.pallas_call(
        _gen_body,
        out_shape=jax.ShapeDtypeStruct((_N, _D), f32),
        in_specs=[vmem] * 8 + [smem],
        out_specs=pl.BlockSpec(memory_space=pltpu.VMEM),
        scratch_shapes=[
            pltpu.VMEM((_N, 3 * _D), f32),
        ],
    )(
        z.astype(f32),
        W1.astype(f32),
        we4_[0:2, :],
        gwcat,
        garm,
        bias_mean,
        dbig,
        der,
        consts,
    )
